# baseline (speedup 1.0000x reference)
import functools

import jax
import jax.numpy as jnp
from jax import lax
from jax.experimental import pallas as pl
from jax.experimental.pallas import tpu as pltpu
from jax.experimental.pallas import tpu_sc as plsc

D = 64
TD = 2 * D
E = 16384
NC = 2
NS = 16
L = 16
NW = NC * NS
EPW = E // NW
CH = 128
NCH = EPW // CH

_mesh = plsc.VectorSubcoreMesh(core_axis_name="c", subcore_axis_name="s")


@functools.partial(
    pl.kernel,
    mesh=_mesh,
    out_type=jax.ShapeDtypeStruct((E,), jnp.float32),
    scratch_types=[
        pltpu.VMEM((EPW,), jnp.int32),
        pltpu.VMEM((EPW,), jnp.int32),
        pltpu.VMEM((2, CH, TD), jnp.float32),
        pltpu.VMEM((2, CH, TD), jnp.float32),
        pltpu.VMEM((TD,), jnp.float32),
        pltpu.VMEM((EPW,), jnp.float32),
        pltpu.VMEM((CH * L,), jnp.float32),
        pltpu.SemaphoreType.DMA,
        pltpu.SemaphoreType.DMA,
        pltpu.SemaphoreType.DMA,
        pltpu.SemaphoreType.DMA,
    ],
    compiler_params=pltpu.CompilerParams(needs_layout_passes=False),
)
def _complex_score(table, rel_hbm, eidx_hbm, out_hbm,
                   hidx_v, tidx_v, hrows, trows, rel_v, out_v, red_v,
                   sem_h0, sem_t0, sem_h1, sem_t1):
    wid = lax.axis_index("s") * NC + lax.axis_index("c")
    base = wid * EPW

    pltpu.sync_copy(eidx_hbm.at[0, pl.ds(base, EPW)], hidx_v)
    pltpu.sync_copy(eidx_hbm.at[1, pl.ds(base, EPW)], tidx_v)

    sems = [(sem_h0, sem_t0), (sem_h1, sem_t1)]

    def start(j, bs):
        sh, st = sems[bs]
        pltpu.async_copy(
            table.at[hidx_v.at[pl.ds(j * CH, CH)]], hrows.at[bs], sh)
        pltpu.async_copy(
            table.at[tidx_v.at[pl.ds(j * CH, CH)]], trows.at[bs], st)

    def wait(j, bs):
        sh, st = sems[bs]
        pltpu.make_async_copy(
            table.at[hidx_v.at[pl.ds(j * CH, CH)]], hrows.at[bs], sh).wait()
        pltpu.make_async_copy(
            table.at[tidx_v.at[pl.ds(j * CH, CH)]], trows.at[bs], st).wait()

    start(0, 0)

    pltpu.sync_copy(rel_hbm.at[0], rel_v)
    rre = [rel_v[pl.ds(k * L, L)] for k in range(D // L)]
    rim = [rel_v[pl.ds(D + k * L, L)] for k in range(D // L)]
    lane = lax.iota(jnp.int32, L)

    def chunk(j, _):
        b = lax.rem(j, 2)

        @pl.when(jnp.logical_and(j + 1 < NCH, b == 0))
        def _():
            start(j + 1, 1)

        @pl.when(jnp.logical_and(j + 1 < NCH, b == 1))
        def _():
            start(j + 1, 0)

        @pl.when(b == 0)
        def _():
            wait(j, 0)

        @pl.when(b == 1)
        def _():
            wait(j, 1)

        def body(g):
            rbase = g * L * L
            for h in range(2):
                accs = []
                for i2 in range(8):
                    i = h * 8 + i2
                    e = g * L + i
                    acc_a = None
                    acc_b = None
                    for k in range(D // L):
                        hre = hrows[b, e, pl.ds(k * L, L)]
                        him = hrows[b, e, pl.ds(D + k * L, L)]
                        tre = trows[b, e, pl.ds(k * L, L)]
                        tim = trows[b, e, pl.ds(D + k * L, L)]
                        ta = hre * (rre[k] * tre + rim[k] * tim)
                        tb = him * (rre[k] * tim - rim[k] * tre)
                        acc_a = ta if acc_a is None else acc_a + ta
                        acc_b = tb if acc_b is None else acc_b + tb
                    accs.append(acc_a + acc_b)
                for i2 in range(8):
                    red_v[pl.ds(rbase + (h * 8 + i2) * L, L)] = accs[i2]
            vbase = rbase + lane * L
            cols = [plsc.load_gather(red_v, [vbase + t]) for t in range(L)]
            while len(cols) > 1:
                cols = [cols[i2] + cols[i2 + 1]
                        for i2 in range(0, len(cols), 2)]
            out_v[pl.ds(j * CH + g * L, L)] = cols[0]

        plsc.parallel_loop(0, CH // L)(body)
        return 0

    lax.fori_loop(0, NCH, chunk, 0)

    pltpu.sync_copy(out_v, out_hbm.at[pl.ds(base, EPW)])


def kernel(node_emb, rel_emb, edge_label_index):
    return _complex_score(node_emb, rel_emb,
                          edge_label_index.astype(jnp.int32))


# speedup vs baseline: 1.3640x; 1.3640x over previous
import functools

import jax
import jax.numpy as jnp
from jax import lax
from jax.experimental import pallas as pl
from jax.experimental.pallas import tpu as pltpu
from jax.experimental.pallas import tpu_sc as plsc

D = 64
TD = 2 * D
E = 16384
NC = 2
NS = 16
L = 16
NW = NC * NS
EPW = E // NW
CH = 128
NCH = EPW // CH

_mesh = plsc.VectorSubcoreMesh(core_axis_name="c", subcore_axis_name="s")


@functools.partial(
    pl.kernel,
    mesh=_mesh,
    out_type=jax.ShapeDtypeStruct((E,), jnp.float32),
    scratch_types=[
        pltpu.VMEM((EPW,), jnp.int32),
        pltpu.VMEM((EPW,), jnp.int32),
        pltpu.VMEM((2, CH, TD), jnp.float32),
        pltpu.VMEM((2, CH, TD), jnp.float32),
        pltpu.VMEM((TD,), jnp.float32),
        pltpu.VMEM((EPW,), jnp.float32),
        pltpu.VMEM((L * L,), jnp.float32),
        pltpu.SemaphoreType.DMA,
        pltpu.SemaphoreType.DMA,
        pltpu.SemaphoreType.DMA,
        pltpu.SemaphoreType.DMA,
    ],
    compiler_params=pltpu.CompilerParams(needs_layout_passes=False),
)
def _complex_score(table, rel_hbm, eidx_hbm, out_hbm,
                   hidx_v, tidx_v, hrows, trows, rel_v, out_v, red_v,
                   sem_h0, sem_t0, sem_h1, sem_t1):
    wid = lax.axis_index("s") * NC + lax.axis_index("c")
    base = wid * EPW

    pltpu.sync_copy(eidx_hbm.at[0, pl.ds(base, EPW)], hidx_v)
    pltpu.sync_copy(eidx_hbm.at[1, pl.ds(base, EPW)], tidx_v)

    sems = [(sem_h0, sem_t0), (sem_h1, sem_t1)]

    def start(j, bs):
        sh, st = sems[bs]
        pltpu.async_copy(
            table.at[hidx_v.at[pl.ds(j * CH, CH)]], hrows.at[bs], sh)
        pltpu.async_copy(
            table.at[tidx_v.at[pl.ds(j * CH, CH)]], trows.at[bs], st)

    def wait(j, bs):
        sh, st = sems[bs]
        pltpu.make_async_copy(
            table.at[hidx_v.at[pl.ds(j * CH, CH)]], hrows.at[bs], sh).wait()
        pltpu.make_async_copy(
            table.at[tidx_v.at[pl.ds(j * CH, CH)]], trows.at[bs], st).wait()

    start(0, 0)

    pltpu.sync_copy(rel_hbm.at[0], rel_v)
    rre = [rel_v[pl.ds(k * L, L)] for k in range(D // L)]
    rim = [rel_v[pl.ds(D + k * L, L)] for k in range(D // L)]
    lane = lax.iota(jnp.int32, L)

    def chunk(j, _):
        b = lax.rem(j, 2)

        @pl.when(jnp.logical_and(j + 1 < NCH, b == 0))
        def _():
            start(j + 1, 1)

        @pl.when(jnp.logical_and(j + 1 < NCH, b == 1))
        def _():
            start(j + 1, 0)

        @pl.when(b == 0)
        def _():
            wait(j, 0)

        @pl.when(b == 1)
        def _():
            wait(j, 1)

        def body(g, _):
            for h in range(2):
                accs = []
                for i2 in range(8):
                    i = h * 8 + i2
                    e = g * L + i
                    acc_a = None
                    acc_b = None
                    for k in range(D // L):
                        hre = hrows[b, e, pl.ds(k * L, L)]
                        him = hrows[b, e, pl.ds(D + k * L, L)]
                        tre = trows[b, e, pl.ds(k * L, L)]
                        tim = trows[b, e, pl.ds(D + k * L, L)]
                        ta = hre * (rre[k] * tre + rim[k] * tim)
                        tb = him * (rre[k] * tim - rim[k] * tre)
                        acc_a = ta if acc_a is None else acc_a + ta
                        acc_b = tb if acc_b is None else acc_b + tb
                    accs.append(acc_a + acc_b)
                for i2 in range(8):
                    red_v[pl.ds((h * 8 + i2) * L, L)] = accs[i2]
            vbase = lane * L
            cols = [plsc.load_gather(red_v, [vbase + t]) for t in range(L)]
            while len(cols) > 1:
                cols = [cols[i2] + cols[i2 + 1]
                        for i2 in range(0, len(cols), 2)]
            out_v[pl.ds(j * CH + g * L, L)] = cols[0]
            return 0

        lax.fori_loop(0, CH // L, body, 0)
        return 0

    lax.fori_loop(0, NCH, chunk, 0)

    pltpu.sync_copy(out_v, out_hbm.at[pl.ds(base, EPW)])


def kernel(node_emb, rel_emb, edge_label_index):
    return _complex_score(node_emb, rel_emb,
                          edge_label_index.astype(jnp.int32))


# speedup vs baseline: 1.3805x; 1.0121x over previous
import functools

import jax
import jax.numpy as jnp
from jax import lax
from jax.experimental import pallas as pl
from jax.experimental.pallas import tpu as pltpu
from jax.experimental.pallas import tpu_sc as plsc

D = 64
TD = 2 * D
E = 16384
NC = 2
NS = 16
L = 16
NW = NC * NS
EPW = E // NW
CH = 128
NCH = EPW // CH

_mesh = plsc.VectorSubcoreMesh(core_axis_name="c", subcore_axis_name="s")


@functools.partial(
    pl.kernel,
    mesh=_mesh,
    out_type=jax.ShapeDtypeStruct((E,), jnp.float32),
    scratch_types=[
        pltpu.VMEM((EPW,), jnp.int32),
        pltpu.VMEM((EPW,), jnp.int32),
        pltpu.VMEM((2, CH, TD), jnp.float32),
        pltpu.VMEM((2, CH, TD), jnp.float32),
        pltpu.VMEM((TD,), jnp.float32),
        pltpu.VMEM((EPW,), jnp.float32),
        pltpu.VMEM((L * L,), jnp.float32),
        pltpu.SemaphoreType.DMA,
        pltpu.SemaphoreType.DMA,
        pltpu.SemaphoreType.DMA,
        pltpu.SemaphoreType.DMA,
    ],
    compiler_params=pltpu.CompilerParams(needs_layout_passes=False),
)
def _complex_score(table, rel_hbm, eidx_hbm, out_hbm,
                   hidx_v, tidx_v, hrows, trows, rel_v, out_v, red_v,
                   sem_h0, sem_t0, sem_h1, sem_t1):
    wid = lax.axis_index("s") * NC + lax.axis_index("c")
    base = wid * EPW

    cp_hi = pltpu.async_copy(eidx_hbm.at[0, pl.ds(base, EPW)], hidx_v, sem_h0)
    cp_ti = pltpu.async_copy(eidx_hbm.at[1, pl.ds(base, EPW)], tidx_v, sem_t0)
    cp_hi.wait()
    cp_ti.wait()

    sems = [(sem_h0, sem_t0), (sem_h1, sem_t1)]

    def start(j, bs):
        sh, st = sems[bs]
        pltpu.async_copy(
            table.at[hidx_v.at[pl.ds(j * CH, CH)]], hrows.at[bs], sh)
        pltpu.async_copy(
            table.at[tidx_v.at[pl.ds(j * CH, CH)]], trows.at[bs], st)

    def wait(j, bs):
        sh, st = sems[bs]
        pltpu.make_async_copy(
            table.at[hidx_v.at[pl.ds(j * CH, CH)]], hrows.at[bs], sh).wait()
        pltpu.make_async_copy(
            table.at[tidx_v.at[pl.ds(j * CH, CH)]], trows.at[bs], st).wait()

    start(0, 0)

    pltpu.sync_copy(rel_hbm.at[0], rel_v)
    rre = [rel_v[pl.ds(k * L, L)] for k in range(D // L)]
    rim = [rel_v[pl.ds(D + k * L, L)] for k in range(D // L)]
    lane = lax.iota(jnp.int32, L)

    def chunk(j, _):
        b = lax.rem(j, 2)

        @pl.when(jnp.logical_and(j + 1 < NCH, b == 0))
        def _():
            start(j + 1, 1)

        @pl.when(jnp.logical_and(j + 1 < NCH, b == 1))
        def _():
            start(j + 1, 0)

        @pl.when(b == 0)
        def _():
            wait(j, 0)

        @pl.when(b == 1)
        def _():
            wait(j, 1)

        def body(g, _):
            for h in range(2):
                accs = []
                for i2 in range(8):
                    i = h * 8 + i2
                    e = g * L + i
                    acc_a = None
                    acc_b = None
                    for k in range(D // L):
                        hre = hrows[b, e, pl.ds(k * L, L)]
                        him = hrows[b, e, pl.ds(D + k * L, L)]
                        tre = trows[b, e, pl.ds(k * L, L)]
                        tim = trows[b, e, pl.ds(D + k * L, L)]
                        ta = hre * (rre[k] * tre + rim[k] * tim)
                        tb = him * (rre[k] * tim - rim[k] * tre)
                        acc_a = ta if acc_a is None else acc_a + ta
                        acc_b = tb if acc_b is None else acc_b + tb
                    accs.append(acc_a + acc_b)
                for i2 in range(8):
                    red_v[pl.ds((h * 8 + i2) * L, L)] = accs[i2]
            vbase = lane * L
            cols = [plsc.load_gather(red_v, [vbase + t]) for t in range(L)]
            while len(cols) > 1:
                cols = [cols[i2] + cols[i2 + 1]
                        for i2 in range(0, len(cols), 2)]
            out_v[pl.ds(j * CH + g * L, L)] = cols[0]
            return 0

        lax.fori_loop(0, CH // L, body, 0)
        return 0

    lax.fori_loop(0, NCH, chunk, 0)

    pltpu.sync_copy(out_v, out_hbm.at[pl.ds(base, EPW)])


def kernel(node_emb, rel_emb, edge_label_index):
    return _complex_score(node_emb, rel_emb,
                          edge_label_index.astype(jnp.int32))
